# Initial kernel scaffold; baseline (speedup 1.0000x reference)
import jax
import jax.numpy as jnp
from jax.experimental import pallas as pl


def kernel(height_distribution, z_prob, bev_ind):
    raise NotImplementedError("write your pallas kernel here")


# speedup vs baseline: 4.7498x; 4.7498x over previous
import functools

import jax
import jax.numpy as jnp
from jax import lax
from jax.experimental import pallas as pl
from jax.experimental.pallas import tpu as pltpu
from jax.experimental.pallas import tpu_sc as plsc

H = 1024
W = 1024
C = 16
HW = H * W
N = 200000
LAMBDA_PSEUDO = 0.001
LOSS_WEIGHT = 1.0
EPS = 1e-9

NW = 32
CHUNK = 6272
N_PAD = NW * CHUNK
SUB = 4
SC_B = CHUNK // SUB
ZB = 8192
SL = HW // 16

_mesh = plsc.VectorSubcoreMesh(core_axis_name="c", subcore_axis_name="s")
_sc_params = pltpu.CompilerParams(use_tc_tiling_on_sc=False)


@functools.partial(
    pl.kernel,
    compiler_params=_sc_params,
    out_type=(
        jax.ShapeDtypeStruct((N_PAD, C), jnp.float32),
        jax.ShapeDtypeStruct((HW,), jnp.float32),
        jax.ShapeDtypeStruct((HW,), jnp.float32),
    ),
    mesh=_mesh,
    scratch_types=(
        pltpu.VMEM((CHUNK,), jnp.int32),
        pltpu.VMEM((CHUNK,), jnp.float32),
        pltpu.VMEM((SC_B, C), jnp.float32),
        pltpu.VMEM((ZB,), jnp.float32),
        pltpu.VMEM_SHARED((HW,), jnp.float32),
        pltpu.SemaphoreType.DMA,
    ),
)
def _sc_gather_count(pos_hbm, vals_hbm, hd_hbm, g_hbm, cnt0_hbm, cnt1_hbm,
                     idx_v, vals_v, rows_v, zbuf, cnt_sh, sem):
    cid = lax.axis_index("c")
    sid = lax.axis_index("s")
    wid = sid * 2 + cid
    base = wid * CHUNK

    pltpu.sync_copy(pos_hbm.at[pl.ds(base, CHUNK)], idx_v)
    pltpu.sync_copy(vals_hbm.at[pl.ds(base, CHUNK)], vals_v)

    @pl.loop(0, SUB)
    def _(t):
        pltpu.async_copy(hd_hbm.at[idx_v.at[pl.ds(t * SC_B, SC_B)]],
                         rows_v, sem).wait()
        pltpu.sync_copy(rows_v, g_hbm.at[pl.ds(base + t * SC_B, SC_B)])

    @pl.loop(0, ZB // 16)
    def _(i):
        zbuf[pl.ds(i * 16, 16)] = jnp.zeros((16,), jnp.float32)

    @pl.loop(0, SL // ZB)
    def _(j):
        pltpu.sync_copy(zbuf, cnt_sh.at[pl.ds(sid * SL + j * ZB, ZB)])

    plsc.subcore_barrier()

    pltpu.sync_copy(vals_v, cnt_sh.at[idx_v], add=True)

    plsc.subcore_barrier()

    @pl.when(cid == 0)
    def _():
        @pl.loop(0, SL // ZB)
        def _(j):
            off = sid * SL + j * ZB
            pltpu.sync_copy(cnt_sh.at[pl.ds(off, ZB)], zbuf)
            pltpu.sync_copy(zbuf, cnt0_hbm.at[pl.ds(off, ZB)])

    @pl.when(cid == 1)
    def _():
        @pl.loop(0, SL // ZB)
        def _(j):
            off = sid * SL + j * ZB
            pltpu.sync_copy(cnt_sh.at[pl.ds(off, ZB)], zbuf)
            pltpu.sync_copy(zbuf, cnt1_hbm.at[pl.ds(off, ZB)])


@functools.partial(
    pl.kernel,
    compiler_params=_sc_params,
    out_type=jax.ShapeDtypeStruct((N_PAD,), jnp.float32),
    mesh=_mesh,
    scratch_types=(
        pltpu.VMEM((CHUNK,), jnp.int32),
        pltpu.VMEM((CHUNK,), jnp.float32),
        pltpu.VMEM((CHUNK,), jnp.float32),
        pltpu.SemaphoreType.DMA,
    ),
)
def _sc_count_to_weight(pos_hbm, cnt0_hbm, cnt1_hbm, w_hbm,
                        idx_v, c0_v, c1_v, sem):
    cid = lax.axis_index("c")
    sid = lax.axis_index("s")
    wid = sid * 2 + cid
    base = wid * CHUNK

    pltpu.sync_copy(pos_hbm.at[pl.ds(base, CHUNK)], idx_v)
    pltpu.async_copy(cnt0_hbm.at[idx_v], c0_v, sem).wait()
    pltpu.async_copy(cnt1_hbm.at[idx_v], c1_v, sem).wait()

    @pl.loop(0, CHUNK // 16)
    def _(k):
        s = pl.ds(k * 16, 16)
        c0_v[s] = 1.0 / (c0_v[s] + c1_v[s])

    pltpu.sync_copy(c0_v, w_hbm.at[pl.ds(base, CHUNK)])


def _tc_total_logsum(hd2):
    def body(x_ref, o_ref):
        @pl.when(pl.program_id(0) == 0)
        def _():
            o_ref[0, 0] = 0.0
        o_ref[0, 0] += jnp.sum(jnp.log(x_ref[...] + EPS))

    return pl.pallas_call(
        body,
        grid=(64,),
        in_specs=[pl.BlockSpec((16, 16384), lambda i: (i, 0))],
        out_specs=pl.BlockSpec(block_shape=(1, 1), index_map=lambda i: (0, 0),
                               memory_space=pltpu.SMEM),
        out_shape=jax.ShapeDtypeStruct((1, 1), jnp.float32),
    )(hd2)


def _tc_point_terms(g2, z2, w2):
    def body(g_ref, z_ref, w_ref, t_ref, m_ref, n_ref):
        @pl.when(pl.program_id(0) == 0)
        def _():
            t_ref[0, 0] = 0.0
            m_ref[0, 0] = 0.0
            n_ref[0, 0] = 0.0
        lg = jnp.log(g_ref[...] + EPS)
        t_ref[0, 0] += jnp.sum(z_ref[...] * lg)
        lane = lax.broadcasted_iota(jnp.int32, (8, 128), 1)
        row = lax.broadcasted_iota(jnp.int32, (8, 128), 0)
        expand = jnp.where(lane // 16 == row, 1.0, 0.0)
        wexp = jnp.dot(w_ref[...], expand, preferred_element_type=jnp.float32)
        m_ref[0, 0] += jnp.sum(wexp * lg)
        n_ref[0, 0] += jnp.sum(w_ref[...])

    scalar_out = pl.BlockSpec(block_shape=(1, 1), index_map=lambda i: (0, 0),
                              memory_space=pltpu.SMEM)
    return pl.pallas_call(
        body,
        grid=(125,),
        in_specs=[
            pl.BlockSpec((200, 128), lambda i: (i, 0)),
            pl.BlockSpec((200, 128), lambda i: (i, 0)),
            pl.BlockSpec((200, 8), lambda i: (i, 0)),
        ],
        out_specs=(scalar_out, scalar_out, scalar_out),
        out_shape=(
            jax.ShapeDtypeStruct((1, 1), jnp.float32),
            jax.ShapeDtypeStruct((1, 1), jnp.float32),
            jax.ShapeDtypeStruct((1, 1), jnp.float32),
        ),
    )(g2, z2, w2)


def kernel(height_distribution, z_prob, bev_ind):
    hd_flat = height_distribution.reshape(HW, C)
    pos = bev_ind[:, 0] * W + bev_ind[:, 1]
    pos_pad = jnp.concatenate(
        [pos, jnp.broadcast_to(pos[N - 1:N], (N_PAD - N,))])
    vals = jnp.concatenate(
        [jnp.ones((N,), jnp.float32), jnp.zeros((N_PAD - N,), jnp.float32)])

    g, cnt0, cnt1 = _sc_gather_count(pos_pad, vals, hd_flat)
    w = _sc_count_to_weight(pos_pad, cnt0, cnt1)

    total_s = _tc_total_logsum(height_distribution.reshape(H, W * C))[0, 0]
    t_sum, m_sum, w_sum = _tc_point_terms(
        g.reshape(N_PAD // 8, 128), z_prob.reshape(N // 8, 128),
        w.reshape(N_PAD // 8, 8))

    loss_true_mean = -t_sum[0, 0] / N
    num_pseudo = HW - w_sum[0, 0]
    loss_pseudo = -(total_s - m_sum[0, 0]) / C / num_pseudo
    total = LOSS_WEIGHT * (loss_true_mean + LAMBDA_PSEUDO * loss_pseudo)
    return jnp.reshape(total, (1,))
